# Initial kernel scaffold; baseline (speedup 1.0000x reference)
import jax
import jax.numpy as jnp
from jax.experimental import pallas as pl


def kernel(general_states, shuf_general_states, intro_states, shuf_intro_states, method_states, shuf_method_states, experiment_states, shuf_experiment_states, relate_states, shuf_relate_states, params, edge_index_intro, edge_index_method, edge_index_experiment, edge_index_relate):
    raise NotImplementedError("write your pallas kernel here")


# speedup vs baseline: 1.3696x; 1.3696x over previous
import functools
import math

import jax
import jax.numpy as jnp
from jax.experimental import pallas as pl
from jax.experimental.pallas import tpu as pltpu

_N = 10000
_BLK = 1000
_SECTIONS = ('general', 'intro', 'method', 'experiment', 'relate')
_ETYPES = ('intro', 'method', 'experiment', 'relate')


def _matmul_body(x_ref, w_ref, b_ref, o_ref):
    o_ref[0] = (
        jnp.dot(x_ref[0], w_ref[0], preferred_element_type=jnp.float32)
        + b_ref[0]
    )


def _stacked_proj(xs, ws, bs):
    s, n, d = xs.shape
    f = ws.shape[-1]
    grid = (s, n // _BLK)
    bs = bs[:, None, :]
    return pl.pallas_call(
        _matmul_body,
        grid=grid,
        in_specs=[
            pl.BlockSpec((1, _BLK, d), lambda i, j: (i, j, 0)),
            pl.BlockSpec((1, d, f), lambda i, j: (i, 0, 0)),
            pl.BlockSpec((1, 1, f), lambda i, j: (i, 0, 0)),
        ],
        out_specs=pl.BlockSpec((1, _BLK, f), lambda i, j: (i, j, 0)),
        out_shape=jax.ShapeDtypeStruct((s, n, f), jnp.float32),
    )(xs, ws, bs)


def _seg_softmax_agg(att, dst, src, val):
    e = jnp.exp(att)
    s = jax.ops.segment_sum(e, dst, num_segments=_N)
    a = e / (s[dst] + 1e-9)
    return jax.ops.segment_sum(a[:, None] * val[src], dst, num_segments=_N)


def kernel(general_states, shuf_general_states, intro_states, shuf_intro_states, method_states, shuf_method_states, experiment_states, shuf_experiment_states, relate_states, shuf_relate_states, params, edge_index_intro, edge_index_method, edge_index_experiment, edge_index_relate):
    feats = {
        'general': general_states,
        'intro': intro_states,
        'method': method_states,
        'experiment': experiment_states,
        'relate': relate_states,
    }
    edges = {
        'intro': edge_index_intro,
        'method': edge_index_method,
        'experiment': edge_index_experiment,
        'relate': edge_index_relate,
    }

    small = [s for s in _SECTIONS if s != 'relate']
    xs = jnp.stack([feats[s] for s in small])
    ws = jnp.stack([
        jnp.concatenate(
            [params['k_' + s + '_w'].T, params['q_' + s + '_w'].T,
             params['v_' + s + '_w'].T], axis=-1)
        for s in small
    ])
    bs = jnp.stack([
        jnp.concatenate(
            [params['k_' + s + '_b'], params['q_' + s + '_b'],
             params['v_' + s + '_b']], axis=-1)
        for s in small
    ])
    kqv_small = _stacked_proj(xs, ws, bs)

    w_rel = jnp.concatenate(
        [params['k_relate_w'].T, params['q_relate_w'].T,
         params['v_relate_w'].T], axis=-1)[None]
    b_rel = jnp.concatenate(
        [params['k_relate_b'], params['q_relate_b'],
         params['v_relate_b']], axis=-1)[None]
    kqv_rel = _stacked_proj(relate_states[None], w_rel, b_rel)[0]

    kqv = {}
    for i, s in enumerate(small):
        kqv['k_' + s] = kqv_small[i, :, 0:64]
        kqv['q_' + s] = kqv_small[i, :, 64:128]
        kqv['v_' + s] = kqv_small[i, :, 128:192]
    kqv['k_relate'] = kqv_rel[:, 0:192]
    kqv['q_relate'] = kqv_rel[:, 192:384]
    kqv['v_relate'] = kqv_rel[:, 384:576]

    d_half = 64
    t_general_acc = []
    t_spec = {}
    for sec in _ETYPES:
        ei = edges[sec]
        src, dst = ei[0], ei[1]
        dk = 256 if sec == 'relate' else 128
        qg = kqv['q_general'][dst]
        kg = kqv['k_general'][src]
        qs_ = kqv['q_' + sec][dst]
        ks = kqv['k_' + sec][src]
        att = ((qg * kg).sum(-1) + (qs_ * ks).sum(-1)) / math.sqrt(dk)
        val = jnp.concatenate([kqv['v_general'], kqv['v_' + sec]], axis=-1)
        h = _seg_softmax_agg(att, dst, src, val)
        t_general_acc.append(h[:, :d_half])
        t_spec[sec] = h[:, d_half:]
    t = {'general': sum(t_general_acc) * 0.25}
    t.update(t_spec)

    ts = jnp.stack([t[s] for s in small])
    wo = jnp.stack([params['a_' + s + '_w'].T for s in small])
    bo = jnp.stack([params['a_' + s + '_b'] for s in small])
    out_small = _stacked_proj(ts, wo, bo)
    out_rel = _stacked_proj(
        t['relate'][None], params['a_relate_w'].T[None],
        params['a_relate_b'][None])[0]

    return (out_small[0], out_small[1], out_small[2], out_small[3], out_rel)
